# Initial kernel scaffold; baseline (speedup 1.0000x reference)
import jax
import jax.numpy as jnp
from jax.experimental import pallas as pl


def kernel(h_V_s, h_V_v, edge_index, h_E_s, h_E_v, seq, params):
    raise NotImplementedError("write your pallas kernel here")


# speedup vs baseline: 4.8808x; 4.8808x over previous
import functools

import jax
import jax.numpy as jnp
from jax import lax
from jax.experimental import pallas as pl
from jax.experimental.pallas import tpu as pltpu
from jax.experimental.pallas import tpu_sc as plsc

N_NODES_IN = 10000
N_EDGES_IN = 160000
N2 = 10240
E2 = 163840
CW = 256
NS = 100
NV = 16
EPS = 1e-8


def _vn3(x2):
    return jnp.sqrt(jnp.maximum(x2, EPS))


def _sca_ln(s, g, b):
    mu = jnp.mean(s, axis=-1, keepdims=True)
    var = jnp.mean((s - mu) ** 2, axis=-1, keepdims=True)
    return (s - mu) / jnp.sqrt(var + 1e-5) * g + b


def _vec_ln(v):
    ssq = jnp.maximum(v[0] ** 2 + v[1] ** 2 + v[2] ** 2, EPS)
    den = jnp.sqrt(jnp.mean(ssq, axis=-1, keepdims=True))
    return [vd / den for vd in v]


def _gvp_tail(vh, wv, gate):
    vo = [vhd @ wv for vhd in vh]
    if gate:
        g = jax.nn.sigmoid(_vn3(vo[0] ** 2 + vo[1] ** 2 + vo[2] ** 2))
        vo = [vod * g for vod in vo]
    return vo


def _store_packed(out_ref, s, v, tail):
    out_ref[:, 0:NS] = s
    for d in range(3):
        out_ref[:, NS + NV * d:NS + NV * (d + 1)] = v[d]
    out_ref[:, 148:CW] = tail


def _node_embed_body(hvs_ref, seq_ref, hvv_ref, lng_ref, lnb_ref, wh_ref,
                     ws1_ref, ws2_ref, ws3_ref, b_ref, wv_ref, out_ref):
    hvs = hvs_ref[...]
    seq = seq_ref[...]
    tot = 1286.0
    mu = (jnp.sum(hvs, 1, keepdims=True) + jnp.sum(seq, 1, keepdims=True)) / tot
    var = (jnp.sum((hvs - mu) ** 2, 1, keepdims=True)
           + jnp.sum((seq - mu) ** 2, 1, keepdims=True)) / tot
    sd = jnp.sqrt(var + 1e-5)
    g = lng_ref[...]
    b = lnb_ref[...]
    a = (hvs - mu) / sd * g[:, :6] + b[:, :6]
    q = (seq - mu) / sd * g[:, 6:] + b[:, 6:]
    v = [hvv_ref[:, 3 * d:3 * d + 3] for d in range(3)]
    v = _vec_ln(v)
    vh = [vd @ wh_ref[...] for vd in v]
    vn = _vn3(vh[0] ** 2 + vh[1] ** 2 + vh[2] ** 2)
    s = a @ ws1_ref[...] + q @ ws2_ref[...] + vn @ ws3_ref[...] + b_ref[...]
    wv = wv_ref[...]
    _store_packed(out_ref, s, [vh[d] @ wv for d in range(3)],
                  jnp.zeros((s.shape[0], CW - 148), jnp.float32))


def _edge_embed_body(hes_ref, hev_ref, lng_ref, lnb_ref, wh00_ref,
                     ws1_ref, ws2_ref, b_ref, wv00_ref, out_ref):
    s = _sca_ln(hes_ref[...], lng_ref[...], lnb_ref[...])
    hev = hev_ref[...]
    den = jnp.sqrt(jnp.maximum(
        hev[:, 0:1] ** 2 + hev[:, 1:2] ** 2 + hev[:, 2:3] ** 2, EPS))
    ev = hev / den
    vh = ev * wh00_ref[0, 0]
    vn = jnp.sqrt(jnp.maximum(
        vh[:, 0:1] ** 2 + vh[:, 1:2] ** 2 + vh[:, 2:3] ** 2, EPS))
    es = s @ ws1_ref[...] + vn * ws2_ref[...] + b_ref[...]
    out_ref[:, 0:32] = es
    out_ref[:, 32:35] = vh * wv00_ref[0, 0]
    out_ref[:, 35:64] = jnp.zeros((es.shape[0], 29), jnp.float32)


def _message_body(g_ref, ef_ref,
                  wha_ref, whm_ref, whb_ref, ws1_ref, ws2_ref, ws3_ref,
                  ws4_ref, b0_ref, wv0_ref,
                  wh1_ref, ws11_ref, ws12_ref, b1_ref, wv1_ref,
                  wh2_ref, ws21_ref, ws22_ref, b2_ref, wv2_ref,
                  out_ref):
    gsrc = g_ref[0]
    gdst = g_ref[1]
    ef = ef_ref[...]
    es = ef[:, 0:32]
    ssrc = gsrc[:, 0:NS]
    sdst = gdst[:, 0:NS]
    whm = whm_ref[...]
    vh = []
    for d in range(3):
        lo = NS + NV * d
        vh.append(gsrc[:, lo:lo + NV] @ wha_ref[...]
                  + ef[:, 32 + d:33 + d] * whm
                  + gdst[:, lo:lo + NV] @ whb_ref[...])
    vn = _vn3(vh[0] ** 2 + vh[1] ** 2 + vh[2] ** 2)
    s = (ssrc @ ws1_ref[...] + es @ ws2_ref[...] + sdst @ ws3_ref[...]
         + vn @ ws4_ref[...] + b0_ref[...])
    s = jax.nn.relu(s)
    vo = _gvp_tail(vh, wv0_ref[...], True)
    vh = [vod @ wh1_ref[...] for vod in vo]
    vn = _vn3(vh[0] ** 2 + vh[1] ** 2 + vh[2] ** 2)
    s = jax.nn.relu(s @ ws11_ref[...] + vn @ ws12_ref[...] + b1_ref[...])
    vo = _gvp_tail(vh, wv1_ref[...], True)
    vh = [vod @ wh2_ref[...] for vod in vo]
    vn = _vn3(vh[0] ** 2 + vh[1] ** 2 + vh[2] ** 2)
    s = s @ ws21_ref[...] + vn @ ws22_ref[...] + b2_ref[...]
    vo = _gvp_tail(vh, wv2_ref[...], False)
    bsz = s.shape[0]
    row = pl.program_id(0) * bsz + lax.broadcasted_iota(jnp.int32, (bsz, 1), 0)
    m = (row < N_EDGES_IN).astype(jnp.float32)
    tail = jnp.concatenate(
        [m, jnp.zeros((bsz, CW - 149), jnp.float32)], axis=1)
    _store_packed(out_ref, s * m, [vo[d] * m for d in range(3)], tail)


CB = 16


def _dedup_body(m_ref, d_ref, mo_ref, do_ref):
    iot = lax.broadcasted_iota(jnp.int32, (128, 128), 0)
    jot = lax.broadcasted_iota(jnp.int32, (128, 128), 1)
    for c in range(CB):
        d = d_ref[c:c + 1, :]
        dc = jnp.swapaxes(d, 0, 1)
        eq = dc == d
        prev = jnp.where(eq & (jot < iot), 1.0, 0.0)
        fo = jnp.sum(prev, axis=1, keepdims=True) == 0.0
        cmat = jnp.where(eq & fo, 1.0, 0.0)
        mo_ref[c * 128:(c + 1) * 128, :] = cmat @ m_ref[c * 128:(c + 1) * 128, :]
        do_ref[c:c + 1, :] = jnp.where(jnp.swapaxes(fo, 0, 1), d, N2)


def _node_update_body(t_ref, *refs):
    (a_refs, (n0g_ref, n0b_ref, f0wh_ref, f0ws1_ref, f0ws2_ref,
              f0b_ref, f0wv_ref, f1wh_ref, f1ws1_ref, f1ws2_ref,
              f1b_ref, f1wv_ref, n1g_ref, n1b_ref), out_ref) = (
        refs[:32], refs[32:-1], refs[-1])
    t = t_ref[...]
    agg = a_refs[0][0]
    for ar in a_refs[1:]:
        agg = agg + ar[0]
    cnt = jnp.maximum(agg[:, 148:149], 1.0)
    s = t[:, 0:NS] + agg[:, 0:NS] / cnt
    v = [t[:, NS + NV * d:NS + NV * (d + 1)]
         + agg[:, NS + NV * d:NS + NV * (d + 1)] / cnt for d in range(3)]
    s = _sca_ln(s, n0g_ref[...], n0b_ref[...])
    v = _vec_ln(v)
    vh = [vd @ f0wh_ref[...] for vd in v]
    vn = _vn3(vh[0] ** 2 + vh[1] ** 2 + vh[2] ** 2)
    fs = jax.nn.relu(s @ f0ws1_ref[...] + vn @ f0ws2_ref[...] + f0b_ref[...])
    fv = _gvp_tail(vh, f0wv_ref[...], True)
    vh = [fvd @ f1wh_ref[...] for fvd in fv]
    vn = _vn3(vh[0] ** 2 + vh[1] ** 2 + vh[2] ** 2)
    fs = fs @ f1ws1_ref[...] + vn @ f1ws2_ref[...] + f1b_ref[...]
    fv = _gvp_tail(vh, f1wv_ref[...], False)
    s = _sca_ln(s + fs, n1g_ref[...], n1b_ref[...])
    v = _vec_ln([v[d] + fv[d] for d in range(3)])
    _store_packed(out_ref, s, v,
                  jnp.zeros((s.shape[0], CW - 148), jnp.float32))


def _out_head_body(t_ref, lng_ref, lnb_ref, wh_ref, ws1_ref, ws2_ref, b_ref,
                   out_ref):
    t = t_ref[...]
    s = _sca_ln(t[:, 0:NS], lng_ref[...], lnb_ref[...])
    v = _vec_ln([t[:, NS + NV * d:NS + NV * (d + 1)] for d in range(3)])
    vh = [vd @ wh_ref[...] for vd in v]
    vn = _vn3(vh[0] ** 2 + vh[1] ** 2 + vh[2] ** 2)
    out_ref[...] = jax.nn.relu(
        s @ ws1_ref[...] + vn @ ws2_ref[...] + b_ref[...])


def _full_spec(a):
    return pl.BlockSpec(a.shape, lambda i: tuple(0 for _ in a.shape))


def _tc(body, grid, row_specs, weights, out_specs, out_shape, args, interp=False):
    in_specs = row_specs + [_full_spec(w) for w in weights]
    return pl.pallas_call(
        body, grid=grid, in_specs=in_specs, out_specs=out_specs,
        out_shape=out_shape, interpret=interp)(*args, *weights)


_MESH = dict(core_axis_name="c", subcore_axis_name="s")


def _sc_gather(table, idx):
    nidx = idx.shape[0]
    per_w = nidx // (32 * 128)

    @functools.partial(
        pl.kernel, mesh=plsc.VectorSubcoreMesh(**_MESH),
        out_type=jax.ShapeDtypeStruct((nidx, CW), jnp.float32),
        scratch_types=[pltpu.VMEM((128,), jnp.int32),
                       pltpu.VMEM((128, CW), jnp.float32),
                       pltpu.SemaphoreType.DMA],
    )
    def k(table_hbm, idx_hbm, out_hbm, idx_v, rows_v, sem):
        wid = lax.axis_index("s") * 2 + lax.axis_index("c")

        def body(j, carry):
            chunk = wid * per_w + j
            pltpu.sync_copy(idx_hbm.at[pl.ds(chunk * 128, 128)], idx_v)
            pltpu.async_copy(table_hbm.at[idx_v], rows_v, sem).wait()
            pltpu.sync_copy(rows_v, out_hbm.at[pl.ds(chunk * 128, 128)])
            return carry

        lax.fori_loop(0, per_w, body, 0)

    return k(table, idx)


def _sc_scatter(msgs, dst):
    nchunk = msgs.shape[0] // 128
    per_w = nchunk // 32
    zch = N2 // 128

    @functools.partial(
        pl.kernel, mesh=plsc.VectorSubcoreMesh(**_MESH),
        out_type=jax.ShapeDtypeStruct((32, N2 + 256, CW), jnp.float32),
        scratch_types=[pltpu.VMEM((128, CW), jnp.float32),
                       pltpu.VMEM((128, CW), jnp.float32),
                       pltpu.VMEM((128,), jnp.int32),
                       pltpu.SemaphoreType.DMA],
    )
    def k(m_hbm, d_hbm, out_hbm, buf, gbuf, idx_v, sem):
        cid = lax.axis_index("c")
        sid = lax.axis_index("s")
        wid = sid * 2 + cid
        zero16 = jnp.zeros((16,), jnp.float32)

        def zb(i, carry):
            buf[i // (CW // 16), pl.ds((i % (CW // 16)) * 16, 16)] = zero16
            return carry

        lax.fori_loop(0, 128 * (CW // 16), zb, 0)

        def zc(z, carry):
            pltpu.sync_copy(buf, out_hbm.at[wid].at[pl.ds(z * 128, 128)])
            return carry

        lax.fori_loop(0, zch, zc, 0)

        def body(j, carry):
            chunk = wid * per_w + j
            pltpu.sync_copy(d_hbm.at[pl.ds(chunk * 128, 128)], idx_v)
            pltpu.sync_copy(m_hbm.at[pl.ds(chunk * 128, 128)], buf)
            pltpu.async_copy(out_hbm.at[wid].at[idx_v], gbuf, sem).wait()

            def acc(i, carry2):
                r = i // (CW // 16)
                cc = (i % (CW // 16)) * 16
                buf[r, pl.ds(cc, 16)] = (buf[r, pl.ds(cc, 16)]
                                         + gbuf[r, pl.ds(cc, 16)])
                return carry2

            lax.fori_loop(0, 128 * (CW // 16), acc, 0)
            pltpu.sync_copy(buf, out_hbm.at[wid].at[idx_v])
            return carry

        lax.fori_loop(0, per_w, body, 0)

    return k(msgs, dst)


def _r2(x):
    return x.reshape(1, -1)


def _prep(params):
    P = {}
    g = params['Wv_gvp']
    P['ne'] = [_r2(params['Wv_ln']['g']), _r2(params['Wv_ln']['b']), g['wh'],
               g['ws']['w'][:6], g['ws']['w'][6:1286], g['ws']['w'][1286:],
               _r2(g['ws']['b']), g['wv']]
    e = params['We_gvp']
    P['ee'] = [_r2(params['We_ln']['g']), _r2(params['We_ln']['b']),
               e['wh'].reshape(1, 1), e['ws']['w'][:32], _r2(e['ws']['w'][32]),
               _r2(e['ws']['b']), e['wv'].reshape(1, 1)]
    o = params['Wout_gvp']
    P['out'] = [_r2(params['Wout_ln']['g']), _r2(params['Wout_ln']['b']),
                o['wh'], o['ws']['w'][:100], o['ws']['w'][100:],
                _r2(o['ws']['b'])]
    P['msg'] = []
    P['upd'] = []
    for lp in params['layers']:
        m0, m1, m2 = lp['conv']['m0'], lp['conv']['m1'], lp['conv']['m2']
        P['msg'].append([
            m0['wh'][0:16], _r2(m0['wh'][16]), m0['wh'][17:33],
            m0['ws']['w'][0:100], m0['ws']['w'][100:132],
            m0['ws']['w'][132:232], m0['ws']['w'][232:265],
            _r2(m0['ws']['b']), m0['wv'],
            m1['wh'], m1['ws']['w'][:100], m1['ws']['w'][100:],
            _r2(m1['ws']['b']), m1['wv'],
            m2['wh'], m2['ws']['w'][:100], m2['ws']['w'][100:],
            _r2(m2['ws']['b']), m2['wv'],
        ])
        P['upd'].append([
            _r2(lp['norm0']['g']), _r2(lp['norm0']['b']),
            lp['ff0']['wh'], lp['ff0']['ws']['w'][:100],
            lp['ff0']['ws']['w'][100:], _r2(lp['ff0']['ws']['b']),
            lp['ff0']['wv'],
            lp['ff1']['wh'], lp['ff1']['ws']['w'][:400],
            lp['ff1']['ws']['w'][400:], _r2(lp['ff1']['ws']['b']),
            lp['ff1']['wv'],
            _r2(lp['norm1']['g']), _r2(lp['norm1']['b']),
        ])
    return P


def kernel(h_V_s, h_V_v, edge_index, h_E_s, h_E_v, seq, params):
    P = _prep(params)
    n_pad = N2 - N_NODES_IN
    e_pad = E2 - N_EDGES_IN

    hvs = jnp.pad(h_V_s, ((0, n_pad), (0, 0)))
    seqp = jnp.pad(seq, ((0, n_pad), (0, 0)))
    hvv = jnp.pad(jnp.swapaxes(h_V_v, 1, 2).reshape(N_NODES_IN, 9),
                  ((0, n_pad), (0, 0)))
    hes = jnp.pad(h_E_s, ((0, e_pad), (0, 0)))
    hev = jnp.pad(h_E_v[:, 0, :], ((0, e_pad), (0, 0)))
    ei = jnp.pad(edge_index, ((0, 0), (0, e_pad)))
    idx = ei.reshape(2 * E2)
    dst2d = ei[1].reshape(E2 // 128, 128)

    BN = 1024
    gn = N2 // BN
    BE = 2048
    BM = 1280

    t = _tc(_node_embed_body, (gn,),
            [pl.BlockSpec((BN, 6), lambda i: (i, 0)),
             pl.BlockSpec((BN, 1280), lambda i: (i, 0)),
             pl.BlockSpec((BN, 9), lambda i: (i, 0))],
            P['ne'],
            pl.BlockSpec((BN, CW), lambda i: (i, 0)),
            jax.ShapeDtypeStruct((N2, CW), jnp.float32),
            [hvs, seqp, hvv])

    ef = _tc(_edge_embed_body, (E2 // BE,),
             [pl.BlockSpec((BE, 32), lambda i: (i, 0)),
              pl.BlockSpec((BE, 3), lambda i: (i, 0))],
             P['ee'],
             pl.BlockSpec((BE, 64), lambda i: (i, 0)),
             jax.ShapeDtypeStruct((E2, 64), jnp.float32),
             [hes, hev])

    for li in range(len(P['msg'])):
        gth = _sc_gather(t, idx).reshape(2, E2, CW)
        msgs = _tc(_message_body, (E2 // BM,),
                   [pl.BlockSpec((2, BM, CW), lambda i: (0, i, 0)),
                    pl.BlockSpec((BM, 64), lambda i: (i, 0))],
                   P['msg'][li],
                   pl.BlockSpec((BM, CW), lambda i: (i, 0)),
                   jax.ShapeDtypeStruct((E2, CW), jnp.float32),
                   [gth, ef])
        msgs2, dst2 = _tc(
            _dedup_body, (E2 // (CB * 128),),
            [pl.BlockSpec((CB * 128, CW), lambda i: (i, 0)),
             pl.BlockSpec((CB, 128), lambda i: (i, 0))],
            [],
            [pl.BlockSpec((CB * 128, CW), lambda i: (i, 0)),
             pl.BlockSpec((CB, 128), lambda i: (i, 0))],
            [jax.ShapeDtypeStruct((E2, CW), jnp.float32),
             jax.ShapeDtypeStruct((E2 // 128, 128), jnp.int32)],
            [msgs, dst2d])
        part = _sc_scatter(msgs2, dst2.reshape(E2))
        BU = 256
        t = _tc(_node_update_body, (N2 // BU,),
                [pl.BlockSpec((BU, CW), lambda i: (i, 0))]
                + [pl.BlockSpec((1, BU, CW),
                                functools.partial(lambda k, i: (k, i, 0), k))
                   for k in range(32)],
                P['upd'][li],
                pl.BlockSpec((BU, CW), lambda i: (i, 0)),
                jax.ShapeDtypeStruct((N2, CW), jnp.float32),
                [t] + [part] * 32)

    out = _tc(_out_head_body, (gn,),
              [pl.BlockSpec((BN, CW), lambda i: (i, 0))],
              P['out'],
              pl.BlockSpec((BN, NS), lambda i: (i, 0)),
              jax.ShapeDtypeStruct((N2, NS), jnp.float32),
              [t])
    return out[:N_NODES_IN]
